# baseline (speedup 1.0000x reference)
import functools

import jax
import jax.numpy as jnp
from jax import lax
from jax.experimental import pallas as pl
from jax.experimental.pallas import tpu as pltpu
from jax.experimental.pallas import tpu_sc as plsc

B = 16384
V = 1000000
D = 32
H = 256

NC = 2
NS = 16
NW = NC * NS
BPW = B // NW
CH = 128
NCH = BPW // CH
L = 16
W = 4 * D

CHK = 8192
RG = 33
QR = RG * CHK
SB = 30
Q = SB * CHK


def _relayout_body(u0, u1, u2, u3, i0, i1, i2, i3, outu_r, outi_r):
    eye = (lax.broadcasted_iota(jnp.int32, (W, W), 0)
           == lax.broadcasted_iota(jnp.int32, (W, W), 1)).astype(jnp.float32)
    dn = (((0,), (0,)), ((), ()))
    cu = jnp.concatenate([u0[:], u1[:], u2[:], u3[:]], axis=0)
    ci = jnp.concatenate([i0[:], i1[:], i2[:], i3[:]], axis=0)
    outu_r[:] = lax.dot_general(cu, eye, dn,
                                precision=lax.Precision.DEFAULT,
                                preferred_element_type=jnp.float32)
    outi_r[:] = lax.dot_general(ci, eye, dn,
                                precision=lax.Precision.DEFAULT,
                                preferred_element_type=jnp.float32)


def _relayout(uetT, ietT):
    out4 = jax.ShapeDtypeStruct((QR, W), jnp.float32)
    mk = lambda k: pl.BlockSpec((D, CHK), lambda i, k=k: (0, k * SB + i))
    return pl.pallas_call(
        _relayout_body,
        grid=(RG,),
        in_specs=[mk(0), mk(1), mk(2), mk(3)] * 2,
        out_specs=[
            pl.BlockSpec((CHK, W), lambda i: (i, 0)),
            pl.BlockSpec((CHK, W), lambda i: (i, 0)),
        ],
        out_shape=[out4, out4],
    )(uetT, uetT, uetT, uetT, ietT, ietT, ietT, ietT)


def _sc_gather(user2d, item2d, ubt, ibt, uet4, iet4):
    mesh = plsc.VectorSubcoreMesh(core_axis_name="c", subcore_axis_name="s")

    @functools.partial(
        pl.kernel,
        out_type=(
            jax.ShapeDtypeStruct((B, W), jnp.float32),
            jax.ShapeDtypeStruct((B, W), jnp.float32),
            jax.ShapeDtypeStruct((B,), jnp.float32),
        ),
        mesh=mesh,
        scratch_types=[
            pltpu.VMEM((NCH, CH), jnp.int32),
            pltpu.VMEM((NCH, CH), jnp.int32),
            pltpu.VMEM((NCH, CH), jnp.int32),
            pltpu.VMEM((NCH, CH), jnp.int32),
            pltpu.VMEM((2, CH, W), jnp.float32),
            pltpu.VMEM((2, CH, W), jnp.float32),
            pltpu.VMEM((BPW,), jnp.float32),
            pltpu.VMEM((BPW,), jnp.float32),
            pltpu.SemaphoreType.DMA,
            pltpu.SemaphoreType.DMA,
            pltpu.SemaphoreType.DMA,
        ],
    )
    def k(user_h, item_h, ubt_h, ibt_h, uet_h, iet_h, xu_h, xi_h, wide_h,
          idx_u, idx_i, qidx_u, qidx_i, bufu, bufi, bu, bi, gsem, wsem, bsem):
        wid = lax.axis_index("s") * NC + lax.axis_index("c")
        base = wid * BPW
        pltpu.sync_copy(user_h.at[pl.ds(wid * NCH, NCH)], idx_u)
        pltpu.sync_copy(item_h.at[pl.ds(wid * NCH, NCH)], idx_i)
        bias_copies = []
        for j in range(NCH):
            dst = pl.ds(j * CH, CH)
            bias_copies.append(pltpu.async_copy(ubt_h.at[idx_u.at[j]],
                                                bu.at[dst], bsem))
            bias_copies.append(pltpu.async_copy(ibt_h.at[idx_i.at[j]],
                                                bi.at[dst], bsem))
        q1, q2, q3 = jnp.int32(Q), jnp.int32(2 * Q), jnp.int32(3 * Q)
        zero = jnp.int32(0)

        def _mod_q(v):
            sub = jnp.where(v >= q3, q3,
                            jnp.where(v >= q2, q2,
                                      jnp.where(v >= q1, q1, zero)))
            return v - sub

        for j in range(NCH):
            src_u, dst_u = idx_u.at[j], qidx_u.at[j]
            src_i, dst_i = idx_i.at[j], qidx_i.at[j]
            for t in range(CH // L):
                s = pl.ds(t * L, L)
                dst_u[s] = _mod_q(src_u[s])
                dst_i[s] = _mod_q(src_i[s])
        writes = [None] * NCH
        for j in range(NCH):
            s = j & 1
            if j >= 2:
                writes[j - 2][0].wait()
                writes[j - 2][1].wait()
            gu = pltpu.async_copy(uet_h.at[qidx_u.at[j]], bufu.at[s], gsem)
            gi = pltpu.async_copy(iet_h.at[qidx_i.at[j]], bufi.at[s], gsem)
            gu.wait()
            gi.wait()
            row = pl.ds(base + j * CH, CH)
            writes[j] = (
                pltpu.async_copy(bufu.at[s], xu_h.at[row], wsem),
                pltpu.async_copy(bufi.at[s], xi_h.at[row], wsem),
            )
        for c in bias_copies:
            c.wait()
        for t in range(BPW // L):
            s = pl.ds(t * L, L)
            bu[s] = bu[s] + bi[s]
        pltpu.sync_copy(bu, wide_h.at[pl.ds(base, BPW)])
        for j in range(NCH - 2, NCH):
            writes[j][0].wait()
            writes[j][1].wait()

    return k(user2d, item2d, ubt, ibt, uet4, iet4)


BT = 2048


def _quarter_mask(xw, idx):
    quarter = ((idx >= Q).astype(jnp.int32) + (idx >= 2 * Q).astype(jnp.int32)
               + (idx >= 3 * Q).astype(jnp.int32))
    lane_q = lax.shift_right_logical(
        lax.broadcasted_iota(jnp.int32, (BT, W), 1), jnp.int32(5))
    return xw * (lane_q == quarter).astype(jnp.float32)


def _mlp_body(u_r, i_r, xuw_r, xiw_r, w_r, W1u_r, W1i_r, b1_r, W2_r,
              b2_r, W3_r, b3_r, out_r):
    xu = _quarter_mask(xuw_r[:], u_r[:])
    xi = _quarter_mask(xiw_r[:], i_r[:])
    h = jnp.dot(xu, W1u_r[:], preferred_element_type=jnp.float32)
    h = h + jnp.dot(xi, W1i_r[:], preferred_element_type=jnp.float32)
    h = jnp.maximum(h + b1_r[:], 0.0)
    h = jnp.maximum(
        jnp.dot(h, W2_r[:], preferred_element_type=jnp.float32) + b2_r[:], 0.0)
    out_r[:] = (jnp.dot(h, W3_r[:], preferred_element_type=jnp.float32)
                + w_r[:] + b3_r[:])


def _mlp(u2d, i2d, xuw, xiw, wide2d, W1u, W1i, b1, W2, b2, W3, b3):
    rep = lambda shape: pl.BlockSpec(shape, lambda i: tuple(0 for _ in shape))
    return pl.pallas_call(
        _mlp_body,
        grid=(B // BT,),
        in_specs=[
            pl.BlockSpec((BT, 1), lambda i: (i, 0)),
            pl.BlockSpec((BT, 1), lambda i: (i, 0)),
            pl.BlockSpec((BT, W), lambda i: (i, 0)),
            pl.BlockSpec((BT, W), lambda i: (i, 0)),
            pl.BlockSpec((BT, 1), lambda i: (i, 0)),
            rep((W, H)),
            rep((W, H)),
            rep((H,)),
            rep((H, H // 2)),
            rep((H // 2,)),
            rep((H // 2, 1)),
            rep((1,)),
        ],
        out_specs=pl.BlockSpec((BT, 1), lambda i: (i, 0)),
        out_shape=jax.ShapeDtypeStruct((B, 1), jnp.float32),
    )(u2d, i2d, xuw, xiw, wide2d, W1u, W1i, b1, W2, b2, W3, b3)


def kernel(user, item, user_bias_table, item_bias_table, user_emb_table,
           item_emb_table, W1, b1, W2, b2, W3, b3):
    user = user.astype(jnp.int32)
    item = item.astype(jnp.int32)
    user2d = user.reshape(B // CH, CH)
    item2d = item.reshape(B // CH, CH)
    ubt = user_bias_table.reshape(V)
    ibt = item_bias_table.reshape(V)
    uet4, iet4 = _relayout(user_emb_table.T, item_emb_table.T)
    xuw, xiw, wide = _sc_gather(user2d, item2d, ubt, ibt, uet4, iet4)
    W1u4 = jnp.concatenate([W1[:D]] * 4, axis=0)
    W1i4 = jnp.concatenate([W1[D:]] * 4, axis=0)
    out = _mlp(user.reshape(B, 1), item.reshape(B, 1), xuw, xiw,
               wide.reshape(B, 1), W1u4, W1i4, b1, W2, b2, W3, b3)
    return out.reshape(B)


# speedup vs baseline: 1.0228x; 1.0228x over previous
import functools

import jax
import jax.numpy as jnp
from jax import lax
from jax.experimental import pallas as pl
from jax.experimental.pallas import tpu as pltpu
from jax.experimental.pallas import tpu_sc as plsc

B = 16384
V = 1000000
D = 32
H = 256

NC = 2
NS = 16
NW = NC * NS
BPW = B // NW
CH = 128
NCH = BPW // CH
L = 16
W = 4 * D

CHK = 4096
RG = 62
QR = RG * CHK
SB = 61
Q = SB * CHK


def _relayout_body(u0, u1, u2, u3, i0, i1, i2, i3, outu_r, outi_r):
    eye = (lax.broadcasted_iota(jnp.int32, (W, W), 0)
           == lax.broadcasted_iota(jnp.int32, (W, W), 1)).astype(jnp.float32)
    dn = (((0,), (0,)), ((), ()))
    cu = jnp.concatenate([u0[:], u1[:], u2[:], u3[:]], axis=0)
    ci = jnp.concatenate([i0[:], i1[:], i2[:], i3[:]], axis=0)
    outu_r[:] = lax.dot_general(cu, eye, dn,
                                precision=lax.Precision.DEFAULT,
                                preferred_element_type=jnp.float32)
    outi_r[:] = lax.dot_general(ci, eye, dn,
                                precision=lax.Precision.DEFAULT,
                                preferred_element_type=jnp.float32)


def _relayout(uetT, ietT):
    out4 = jax.ShapeDtypeStruct((QR, W), jnp.float32)
    mk = lambda k: pl.BlockSpec((D, CHK), lambda i, k=k: (0, k * SB + i))
    return pl.pallas_call(
        _relayout_body,
        grid=(RG,),
        in_specs=[mk(0), mk(1), mk(2), mk(3)] * 2,
        out_specs=[
            pl.BlockSpec((CHK, W), lambda i: (i, 0)),
            pl.BlockSpec((CHK, W), lambda i: (i, 0)),
        ],
        out_shape=[out4, out4],
    )(uetT, uetT, uetT, uetT, ietT, ietT, ietT, ietT)


def _sc_gather(user2d, item2d, ubt, ibt, uet4, iet4):
    mesh = plsc.VectorSubcoreMesh(core_axis_name="c", subcore_axis_name="s")

    @functools.partial(
        pl.kernel,
        out_type=(
            jax.ShapeDtypeStruct((B, W), jnp.float32),
            jax.ShapeDtypeStruct((B, W), jnp.float32),
            jax.ShapeDtypeStruct((B,), jnp.float32),
        ),
        mesh=mesh,
        scratch_types=[
            pltpu.VMEM((NCH, CH), jnp.int32),
            pltpu.VMEM((NCH, CH), jnp.int32),
            pltpu.VMEM((NCH, CH), jnp.int32),
            pltpu.VMEM((NCH, CH), jnp.int32),
            pltpu.VMEM((2, CH, W), jnp.float32),
            pltpu.VMEM((2, CH, W), jnp.float32),
            pltpu.VMEM((BPW,), jnp.float32),
            pltpu.VMEM((BPW,), jnp.float32),
            pltpu.SemaphoreType.DMA,
            pltpu.SemaphoreType.DMA,
            pltpu.SemaphoreType.DMA,
        ],
    )
    def k(user_h, item_h, ubt_h, ibt_h, uet_h, iet_h, xu_h, xi_h, wide_h,
          idx_u, idx_i, qidx_u, qidx_i, bufu, bufi, bu, bi, gsem, wsem, bsem):
        wid = lax.axis_index("s") * NC + lax.axis_index("c")
        base = wid * BPW
        pltpu.sync_copy(user_h.at[pl.ds(wid * NCH, NCH)], idx_u)
        pltpu.sync_copy(item_h.at[pl.ds(wid * NCH, NCH)], idx_i)
        bias_copies = []
        for j in range(NCH):
            dst = pl.ds(j * CH, CH)
            bias_copies.append(pltpu.async_copy(ubt_h.at[idx_u.at[j]],
                                                bu.at[dst], bsem))
            bias_copies.append(pltpu.async_copy(ibt_h.at[idx_i.at[j]],
                                                bi.at[dst], bsem))
        q1, q2, q3 = jnp.int32(Q), jnp.int32(2 * Q), jnp.int32(3 * Q)
        zero = jnp.int32(0)

        def _mod_q(v):
            sub = jnp.where(v >= q3, q3,
                            jnp.where(v >= q2, q2,
                                      jnp.where(v >= q1, q1, zero)))
            return v - sub

        for j in range(NCH):
            src_u, dst_u = idx_u.at[j], qidx_u.at[j]
            src_i, dst_i = idx_i.at[j], qidx_i.at[j]
            for t in range(CH // L):
                s = pl.ds(t * L, L)
                dst_u[s] = _mod_q(src_u[s])
                dst_i[s] = _mod_q(src_i[s])
        writes = [None] * NCH
        for j in range(NCH):
            s = j & 1
            if j >= 2:
                writes[j - 2][0].wait()
                writes[j - 2][1].wait()
            gu = pltpu.async_copy(uet_h.at[qidx_u.at[j]], bufu.at[s], gsem)
            gi = pltpu.async_copy(iet_h.at[qidx_i.at[j]], bufi.at[s], gsem)
            gu.wait()
            gi.wait()
            row = pl.ds(base + j * CH, CH)
            writes[j] = (
                pltpu.async_copy(bufu.at[s], xu_h.at[row], wsem),
                pltpu.async_copy(bufi.at[s], xi_h.at[row], wsem),
            )
        for c in bias_copies:
            c.wait()
        for t in range(BPW // L):
            s = pl.ds(t * L, L)
            bu[s] = bu[s] + bi[s]
        pltpu.sync_copy(bu, wide_h.at[pl.ds(base, BPW)])
        for j in range(NCH - 2, NCH):
            writes[j][0].wait()
            writes[j][1].wait()

    return k(user2d, item2d, ubt, ibt, uet4, iet4)


BT = 2048


def _quarter_mask(xw, idx):
    quarter = ((idx >= Q).astype(jnp.int32) + (idx >= 2 * Q).astype(jnp.int32)
               + (idx >= 3 * Q).astype(jnp.int32))
    lane_q = lax.shift_right_logical(
        lax.broadcasted_iota(jnp.int32, (BT, W), 1), jnp.int32(5))
    return xw * (lane_q == quarter).astype(jnp.float32)


def _mlp_body(u_r, i_r, xuw_r, xiw_r, w_r, W1u_r, W1i_r, b1_r, W2_r,
              b2_r, W3_r, b3_r, out_r):
    xu = _quarter_mask(xuw_r[:], u_r[:])
    xi = _quarter_mask(xiw_r[:], i_r[:])
    h = jnp.dot(xu, W1u_r[:], preferred_element_type=jnp.float32)
    h = h + jnp.dot(xi, W1i_r[:], preferred_element_type=jnp.float32)
    h = jnp.maximum(h + b1_r[:], 0.0)
    h = jnp.maximum(
        jnp.dot(h, W2_r[:], preferred_element_type=jnp.float32) + b2_r[:], 0.0)
    out_r[:] = (jnp.dot(h, W3_r[:], preferred_element_type=jnp.float32)
                + w_r[:] + b3_r[:])


def _mlp(u2d, i2d, xuw, xiw, wide2d, W1u, W1i, b1, W2, b2, W3, b3):
    rep = lambda shape: pl.BlockSpec(shape, lambda i: tuple(0 for _ in shape))
    return pl.pallas_call(
        _mlp_body,
        grid=(B // BT,),
        in_specs=[
            pl.BlockSpec((BT, 1), lambda i: (i, 0)),
            pl.BlockSpec((BT, 1), lambda i: (i, 0)),
            pl.BlockSpec((BT, W), lambda i: (i, 0)),
            pl.BlockSpec((BT, W), lambda i: (i, 0)),
            pl.BlockSpec((BT, 1), lambda i: (i, 0)),
            rep((W, H)),
            rep((W, H)),
            rep((H,)),
            rep((H, H // 2)),
            rep((H // 2,)),
            rep((H // 2, 1)),
            rep((1,)),
        ],
        out_specs=pl.BlockSpec((BT, 1), lambda i: (i, 0)),
        out_shape=jax.ShapeDtypeStruct((B, 1), jnp.float32),
    )(u2d, i2d, xuw, xiw, wide2d, W1u, W1i, b1, W2, b2, W3, b3)


def kernel(user, item, user_bias_table, item_bias_table, user_emb_table,
           item_emb_table, W1, b1, W2, b2, W3, b3):
    user = user.astype(jnp.int32)
    item = item.astype(jnp.int32)
    user2d = user.reshape(B // CH, CH)
    item2d = item.reshape(B // CH, CH)
    ubt = user_bias_table.reshape(V)
    ibt = item_bias_table.reshape(V)
    uet4, iet4 = _relayout(user_emb_table.T, item_emb_table.T)
    xuw, xiw, wide = _sc_gather(user2d, item2d, ubt, ibt, uet4, iet4)
    W1u4 = jnp.concatenate([W1[:D]] * 4, axis=0)
    W1i4 = jnp.concatenate([W1[D:]] * 4, axis=0)
    out = _mlp(user.reshape(B, 1), item.reshape(B, 1), xuw, xiw,
               wide.reshape(B, 1), W1u4, W1i4, b1, W2, b2, W3, b3)
    return out.reshape(B)
